# Initial kernel scaffold; baseline (speedup 1.0000x reference)
import jax
import jax.numpy as jnp
from jax.experimental import pallas as pl


def kernel(x, w1_t, b1_2d, w2_p, b2_p):
    raise NotImplementedError("write your pallas kernel here")


# speedup vs baseline: 1.0118x; 1.0118x over previous
import jax
import jax.numpy as jnp
from jax.experimental import pallas as pl
from jax.experimental.pallas import tpu as pltpu

_N_STATES = 4
_HIDDEN = 25
_N_ACT = 2
_PACK = 32
_LANES = _PACK * _N_STATES
_ROW_TILE = 2048
_CHUNK = 128


def _packed_mlp_kernel(x_ref, w1_ref, b1_ref, w2_ref, b2_ref, d_ref,
                       q_ref, a_ref):
    rows = x_ref.shape[0]
    ch = min(_CHUNK, rows)
    for c in range(0, rows, ch):
        xc = x_ref[c:c + ch, :]
        h = jnp.dot(xc, w1_ref[...], preferred_element_type=jnp.float32)
        h = jnp.maximum(h + b1_ref[...], 0.0)
        q = jnp.dot(h, w2_ref[...], preferred_element_type=jnp.float32)
        q = q + b2_ref[...]
        q_ref[c:c + ch, :] = q
        diff = jnp.dot(q, d_ref[...], preferred_element_type=jnp.float32,
                       precision=jax.lax.Precision.HIGHEST)
        a_ref[c:c + ch, :] = (diff > 0.0).astype(jnp.int32)


def kernel(x, w1_t, b1_2d, w2_p, b2_p):
    B = x.shape[0]
    R = B // _PACK
    xr = x.reshape(R, _LANES)

    eye = jnp.eye(_PACK, dtype=jnp.float32)
    w1b = jnp.kron(eye, w1_t)
    b1b = jnp.tile(b1_2d, (1, _PACK))
    w2n = w2_p[:, :_N_ACT]
    b2n = b2_p[:, :_N_ACT]
    w2b = jnp.kron(eye, w2n)
    b2b = jnp.tile(b2n, (1, _PACK))
    dsel = jnp.kron(eye, jnp.array([[-1.0], [1.0]], jnp.float32))

    row_tile = _ROW_TILE if R % _ROW_TILE == 0 else R
    H = _PACK * _HIDDEN
    A = _PACK * _N_ACT

    q_pk, a_pk = pl.pallas_call(
        _packed_mlp_kernel,
        grid=(R // row_tile,),
        in_specs=[
            pl.BlockSpec((row_tile, _LANES), lambda i: (i, 0)),
            pl.BlockSpec((_LANES, H), lambda i: (0, 0)),
            pl.BlockSpec((1, H), lambda i: (0, 0)),
            pl.BlockSpec((H, A), lambda i: (0, 0)),
            pl.BlockSpec((1, A), lambda i: (0, 0)),
            pl.BlockSpec((A, _PACK), lambda i: (0, 0)),
        ],
        out_specs=(
            pl.BlockSpec((row_tile, A), lambda i: (i, 0)),
            pl.BlockSpec((row_tile, _PACK), lambda i: (i, 0)),
        ),
        out_shape=(
            jax.ShapeDtypeStruct((R, A), jnp.float32),
            jax.ShapeDtypeStruct((R, _PACK), jnp.int32),
        ),
        compiler_params=pltpu.CompilerParams(
            dimension_semantics=("parallel",),
        ),
    )(xr, w1b, b1b, w2b, b2b, dsel)

    return q_pk.reshape(B, _N_ACT), a_pk.reshape(B)


# speedup vs baseline: 14.3456x; 14.1778x over previous
import jax
import jax.numpy as jnp
from jax.experimental import pallas as pl
from jax.experimental.pallas import tpu as pltpu

_N_STATES = 4
_HIDDEN = 25
_N_ACT = 2
_LANE_TILE = 16384
_CHUNK = 1024


def _mlp_t_kernel(x_ref, w1t_ref, b1c_ref, w2t_ref, b2c_ref, q_ref, a_ref):
    lanes = x_ref.shape[1]
    ch = min(_CHUNK, lanes)
    w1t = w1t_ref[...]
    b1c = b1c_ref[...]
    w2t = w2t_ref[...]
    b2c = b2c_ref[...]
    for c in range(0, lanes, ch):
        xc = x_ref[:, c:c + ch]
        h = jnp.dot(w1t, xc, preferred_element_type=jnp.float32)
        h = jnp.maximum(h + b1c, 0.0)
        q = jnp.dot(w2t, h, preferred_element_type=jnp.float32) + b2c
        q_ref[:, c:c + ch] = q
        a_ref[:, c:c + ch] = (q[1:2, :] > q[0:1, :]).astype(jnp.int32)


def kernel(x, w1_t, b1_2d, w2_p, b2_p):
    B = x.shape[0]
    xt = x.T
    w1t = w1_t.T
    b1c = b1_2d.T
    w2t = w2_p[:, :_N_ACT].T
    b2c = b2_p[:1, :_N_ACT].T

    lane_tile = _LANE_TILE if B % _LANE_TILE == 0 else B

    q_t, a_t = pl.pallas_call(
        _mlp_t_kernel,
        grid=(B // lane_tile,),
        in_specs=[
            pl.BlockSpec((_N_STATES, lane_tile), lambda i: (0, i)),
            pl.BlockSpec((_HIDDEN, _N_STATES), lambda i: (0, 0)),
            pl.BlockSpec((_HIDDEN, 1), lambda i: (0, 0)),
            pl.BlockSpec((_N_ACT, _HIDDEN), lambda i: (0, 0)),
            pl.BlockSpec((_N_ACT, 1), lambda i: (0, 0)),
        ],
        out_specs=(
            pl.BlockSpec((_N_ACT, lane_tile), lambda i: (0, i)),
            pl.BlockSpec((1, lane_tile), lambda i: (0, i)),
        ),
        out_shape=(
            jax.ShapeDtypeStruct((_N_ACT, B), jnp.float32),
            jax.ShapeDtypeStruct((1, B), jnp.int32),
        ),
        compiler_params=pltpu.CompilerParams(
            dimension_semantics=("parallel",),
        ),
    )(xt, w1t, b1c, w2t, b2c)

    return q_t.T, a_t.reshape(B)


# speedup vs baseline: 32.6684x; 2.2772x over previous
import jax
import jax.numpy as jnp
from jax.experimental import pallas as pl
from jax.experimental.pallas import tpu as pltpu

_N_STATES = 4
_HIDDEN = 25
_N_ACT = 2
_LANE_TILE = 16384
_CHUNK = 1024


def _mlp_t_kernel(x_ref, w1t_ref, b1c_ref, w2t_ref, b2c_ref, q_ref, a_ref):
    lanes = x_ref.shape[1]
    ch = min(_CHUNK, lanes)
    nc = lanes // ch
    w1t = w1t_ref[...]
    b1c = b1c_ref[...]
    w2t = w2t_ref[...]
    b2c = b2c_ref[...]

    def dot1(c):
        xc = x_ref[:, c * ch:(c + 1) * ch]
        return jnp.dot(w1t, xc, preferred_element_type=jnp.float32)

    def dot2(h):
        hr = jnp.maximum(h + b1c, 0.0)
        return jnp.dot(w2t, hr, preferred_element_type=jnp.float32)

    def emit(c, q0):
        q = q0 + b2c
        q_ref[:, c * ch:(c + 1) * ch] = q
        a_ref[:, c * ch:(c + 1) * ch] = (q[1:2, :] > q[0:1, :]).astype(jnp.int32)

    hbuf = [None] * nc
    qbuf = [None] * nc
    for c in range(nc + 4):
        if c < nc:
            hbuf[c] = dot1(c)
        if 2 <= c < nc + 2:
            qbuf[c - 2] = dot2(hbuf[c - 2])
            hbuf[c - 2] = None
        if c >= 4:
            emit(c - 4, qbuf[c - 4])
            qbuf[c - 4] = None


def kernel(x, w1_t, b1_2d, w2_p, b2_p):
    B = x.shape[0]
    xt = x.T
    w1t = w1_t.T
    b1c = b1_2d.T
    w2t = w2_p[:, :_N_ACT].T
    b2c = b2_p[:1, :_N_ACT].T

    lane_tile = _LANE_TILE if B % _LANE_TILE == 0 else B

    q_t, a_t = pl.pallas_call(
        _mlp_t_kernel,
        grid=(B // lane_tile,),
        in_specs=[
            pl.BlockSpec((_N_STATES, lane_tile), lambda i: (0, i)),
            pl.BlockSpec((_HIDDEN, _N_STATES), lambda i: (0, 0)),
            pl.BlockSpec((_HIDDEN, 1), lambda i: (0, 0)),
            pl.BlockSpec((_N_ACT, _HIDDEN), lambda i: (0, 0)),
            pl.BlockSpec((_N_ACT, 1), lambda i: (0, 0)),
        ],
        out_specs=(
            pl.BlockSpec((_N_ACT, lane_tile), lambda i: (0, i)),
            pl.BlockSpec((1, lane_tile), lambda i: (0, i)),
        ),
        out_shape=(
            jax.ShapeDtypeStruct((_N_ACT, B), jnp.float32),
            jax.ShapeDtypeStruct((1, B), jnp.int32),
        ),
        compiler_params=pltpu.CompilerParams(
            dimension_semantics=("parallel",),
        ),
    )(xt, w1t, b1c, w2t, b2c)

    return q_t.T, a_t.reshape(B)


# speedup vs baseline: 62.5810x; 1.9156x over previous
import jax
import jax.numpy as jnp
from jax.experimental import pallas as pl
from jax.experimental.pallas import tpu as pltpu

_N_STATES = 4
_HIDDEN = 25
_N_ACT = 2
_LANE_TILE = 65536
_CHUNK = 2048


def _mlp_t_kernel(x_ref, w1t_ref, b1c_ref, w2t_ref, b2c_ref, q_ref, a_ref):
    lanes = x_ref.shape[1]
    ch = min(_CHUNK, lanes)
    nc = lanes // ch
    w1t = w1t_ref[...]
    b1c = b1c_ref[...]
    w2t = w2t_ref[...]
    b2c = b2c_ref[...]

    def dot1(c):
        xc = x_ref[:, c * ch:(c + 1) * ch]
        return jnp.dot(w1t, xc, preferred_element_type=jnp.float32)

    def dot2(h):
        hr = jnp.maximum(h + b1c, 0.0)
        return jnp.dot(w2t, hr, preferred_element_type=jnp.float32)

    def emit(c, q0):
        q = q0 + b2c
        q_ref[:, c * ch:(c + 1) * ch] = q
        a_ref[:, c * ch:(c + 1) * ch] = (q[1:2, :] > q[0:1, :]).astype(jnp.int32)

    hbuf = [None] * nc
    qbuf = [None] * nc
    for c in range(nc + 4):
        if c < nc:
            hbuf[c] = dot1(c)
        if 2 <= c < nc + 2:
            qbuf[c - 2] = dot2(hbuf[c - 2])
            hbuf[c - 2] = None
        if c >= 4:
            emit(c - 4, qbuf[c - 4])
            qbuf[c - 4] = None


def kernel(x, w1_t, b1_2d, w2_p, b2_p):
    B = x.shape[0]
    xt = x.T
    w1t = w1_t.T
    b1c = b1_2d.T
    w2t = w2_p[:, :_N_ACT].T
    b2c = b2_p[:1, :_N_ACT].T

    lane_tile = _LANE_TILE if B % _LANE_TILE == 0 else B

    q_t, a_t = pl.pallas_call(
        _mlp_t_kernel,
        grid=(B // lane_tile,),
        in_specs=[
            pl.BlockSpec((_N_STATES, lane_tile), lambda i: (0, i)),
            pl.BlockSpec((_HIDDEN, _N_STATES), lambda i: (0, 0)),
            pl.BlockSpec((_HIDDEN, 1), lambda i: (0, 0)),
            pl.BlockSpec((_N_ACT, _HIDDEN), lambda i: (0, 0)),
            pl.BlockSpec((_N_ACT, 1), lambda i: (0, 0)),
        ],
        out_specs=(
            pl.BlockSpec((_N_ACT, lane_tile), lambda i: (0, i)),
            pl.BlockSpec((1, lane_tile), lambda i: (0, i)),
        ),
        out_shape=(
            jax.ShapeDtypeStruct((_N_ACT, B), jnp.float32),
            jax.ShapeDtypeStruct((1, B), jnp.int32),
        ),
        compiler_params=pltpu.CompilerParams(
            dimension_semantics=("parallel",),
        ),
    )(xt, w1t, b1c, w2t, b2c)

    return q_t.T, a_t.reshape(B)


# speedup vs baseline: 72.0695x; 1.1516x over previous
import jax
import jax.numpy as jnp
from jax.experimental import pallas as pl
from jax.experimental.pallas import tpu as pltpu

_N_STATES = 4
_HIDDEN = 25
_N_ACT = 2
_LANE_TILE = 131072
_CHUNK = 4096


def _mlp_t_kernel(x_ref, w1t_ref, b1c_ref, w2t_ref, b2c_ref, q_ref, a_ref):
    lanes = x_ref.shape[1]
    ch = min(_CHUNK, lanes)
    nc = lanes // ch
    w1t = w1t_ref[...]
    b1c = b1c_ref[...]
    w2t = w2t_ref[...]
    b2c = b2c_ref[...]

    def dot1(c):
        xc = x_ref[:, c * ch:(c + 1) * ch]
        return jnp.dot(w1t, xc, preferred_element_type=jnp.float32)

    def dot2(h):
        hr = jnp.maximum(h + b1c, 0.0)
        return jnp.dot(w2t, hr, preferred_element_type=jnp.float32)

    def emit(c, q0):
        q = q0 + b2c
        q_ref[:, c * ch:(c + 1) * ch] = q
        a_ref[:, c * ch:(c + 1) * ch] = (q[1:2, :] > q[0:1, :]).astype(jnp.int32)

    hbuf = [None] * nc
    qbuf = [None] * nc
    for c in range(nc + 4):
        if c < nc:
            hbuf[c] = dot1(c)
        if 2 <= c < nc + 2:
            qbuf[c - 2] = dot2(hbuf[c - 2])
            hbuf[c - 2] = None
        if c >= 4:
            emit(c - 4, qbuf[c - 4])
            qbuf[c - 4] = None


def kernel(x, w1_t, b1_2d, w2_p, b2_p):
    B = x.shape[0]
    xt = x.T
    w1t = w1_t.T
    b1c = b1_2d.T
    w2t = w2_p[:, :_N_ACT].T
    b2c = b2_p[:1, :_N_ACT].T

    lane_tile = _LANE_TILE if B % _LANE_TILE == 0 else B

    q_t, a_t = pl.pallas_call(
        _mlp_t_kernel,
        grid=(B // lane_tile,),
        in_specs=[
            pl.BlockSpec((_N_STATES, lane_tile), lambda i: (0, i)),
            pl.BlockSpec((_HIDDEN, _N_STATES), lambda i: (0, 0)),
            pl.BlockSpec((_HIDDEN, 1), lambda i: (0, 0)),
            pl.BlockSpec((_N_ACT, _HIDDEN), lambda i: (0, 0)),
            pl.BlockSpec((_N_ACT, 1), lambda i: (0, 0)),
        ],
        out_specs=(
            pl.BlockSpec((_N_ACT, lane_tile), lambda i: (0, i)),
            pl.BlockSpec((1, lane_tile), lambda i: (0, i)),
        ),
        out_shape=(
            jax.ShapeDtypeStruct((_N_ACT, B), jnp.float32),
            jax.ShapeDtypeStruct((1, B), jnp.int32),
        ),
        compiler_params=pltpu.CompilerParams(
            dimension_semantics=("parallel",),
        ),
    )(xt, w1t, b1c, w2t, b2c)

    return q_t.T, a_t.reshape(B)
